# baseline (speedup 1.0000x reference)
import jax
import jax.numpy as jnp
from jax import lax
from jax.experimental import pallas as pl
from jax.experimental.pallas import tpu as pltpu
from jax.experimental.pallas import tpu_sc as plsc

NE = 200000
ND = 128
NNODES = 100000
NW = 32
NT = 3136
NPAD = NW * NT
DUMP = NPAD
GROUPS = NT // 16
TBUF = 3328
NTR = TBUF // 128
ECH1 = 6256
NEPAD = NW * ECH1
EV1 = ECH1 // 16


def _shift_up(x, lane):
    idx = jnp.minimum(lane + 1, 15).reshape(16, 1)
    return lax.gather(
        x, idx,
        dimension_numbers=lax.GatherDimensionNumbers(
            offset_dims=(), collapsed_slice_dims=(0,), start_index_map=(0,)),
        slice_sizes=(1,),
        mode=lax.GatherScatterMode.PROMISE_IN_BOUNDS)


def _body1(nid_hbm, partials_out, lp_ref, ev_ref):
    wid = lax.axis_index("s") * 2 + lax.axis_index("c")
    lane = lax.iota(jnp.int32, 16)
    neg1 = jnp.full((16,), -1, jnp.int32)

    def init_lp(g, _):
        for q in range(4):
            lp_ref[pl.ds(g * 64 + q * 16, 16)] = neg1
        return 0
    lax.fori_loop(0, NPAD // 64, init_lp, 0)

    base = wid * ECH1
    pltpu.sync_copy(nid_hbm.at[pl.ds(base, ECH1)], ev_ref)

    def ev_vec(i, _):
        nid = ev_ref[pl.ds(i * 16, 16)]
        pos = base + i * 16 + lane
        key = lax.shift_left(nid, 4) + lane
        skey, spos = lax.sort([key, pos], num_keys=1)
        snid = lax.shift_right_arithmetic(skey, 4)
        nxt = _shift_up(snid, lane)
        is_last = (snid != nxt) | (lane == 15)
        m = is_last & (spos < NE)
        localc = jnp.clip(snid, 0, NPAD - 1)
        plsc.store_scatter(lp_ref, [localc], spos, mask=m)
        return 0
    lax.fori_loop(0, EV1, ev_vec, 0)

    pltpu.sync_copy(lp_ref, partials_out.at[pl.ds(wid * NPAD, NPAD)])


_k1 = pl.kernel(
    _body1,
    out_type=[jax.ShapeDtypeStruct((NW * NPAD,), jnp.int32)],
    mesh=plsc.VectorSubcoreMesh(core_axis_name="c", subcore_axis_name="s"),
    compiler_params=pltpu.CompilerParams(needs_layout_passes=False),
    scratch_types=[
        pltpu.VMEM((NPAD,), jnp.int32),
        pltpu.VMEM((ECH1,), jnp.int32),
    ],
)


def _body2(partials, msg_hbm, ts_hbm, zeros_hbm, nn_hbm,
           msg_out, ts_out, vi_out,
           lp_ref, pb0, pb1, pb2, pb3, pb4, pb5, pb6, pb7,
           nn_ref, ts_idx, vi_buf, idst, idst2, ts_buf,
           rows_a, rows_b, rows_c, rows_d, zbuf, semg, sems, semt):
    pbufs = [pb0, pb1, pb2, pb3, pb4, pb5, pb6, pb7]
    rbufs = [rows_a, rows_b, rows_c, rows_d]
    wid = lax.axis_index("s") * 2 + lax.axis_index("c")
    lo = wid * NT
    lane = lax.iota(jnp.int32, 16)

    pltpu.sync_copy(nn_hbm, nn_ref)
    pltpu.sync_copy(zeros_hbm, zbuf)

    neg1 = jnp.full((16,), -1, jnp.int32)
    zero16 = jnp.zeros((16,), jnp.int32)
    dump16 = jnp.full((16,), DUMP, jnp.int32)

    def init_lp(g, _):
        lp_ref[pl.ds(g * 16, 16)] = neg1
        return 0
    lax.fori_loop(0, GROUPS, init_lp, 0)

    def init_bufs(g, _):
        ts_idx[pl.ds(g * 16, 16)] = zero16
        idst[pl.ds(g * 16, 16)] = dump16
        return 0
    lax.fori_loop(0, TBUF // 16, init_bufs, 0)

    for b in range(4):
        das = [pltpu.async_copy(
                   partials.at[pl.ds((b * 8 + r) * NPAD + lo, NT)],
                   pbufs[r], semt)
               for r in range(8)]
        for d in das:
            d.wait()

        def mg(g, _):
            acc = lp_ref[pl.ds(g * 16, 16)]
            for r in range(8):
                acc = jnp.maximum(acc, pbufs[r][pl.ds(g * 16, 16)])
            lp_ref[pl.ds(g * 16, 16)] = acc
            return 0
        lax.fori_loop(0, GROUPS, mg, 0)

    nn = nn_ref[pl.ds(0, 16)]

    def a_body(g, ni):
        lp = lp_ref[pl.ds(g * 16, 16)]
        node = lo + g * 16 + lane
        valid = (lp >= 0) & (node < nn)
        safe = jnp.maximum(lp, 0)
        ts_idx[pl.ds(g * 16, 16)] = safe
        vi_buf[pl.ds(g * 16, 16)] = jnp.where(valid, 1, 0)
        inv = ~valid
        plsc.store_compressed(idst.at[pl.ds(ni, 16)], node, mask=inv)
        cv = jnp.max(plsc.all_reduce_population_count(valid))
        return ni + (16 - cv)

    ni = lax.fori_loop(0, GROUPS, a_body, jnp.int32(0))

    def c_body(j, _):
        for q in range(8):
            idst2[j, pl.ds(q * 16, 16)] = idst[pl.ds(j * 128 + q * 16, 16)]
        return 0
    lax.fori_loop(0, NTR, c_body, 0)

    def ts_f(j, _):
        pltpu.async_copy(ts_hbm.at[ts_idx.at[pl.ds(j * 128, 128)]],
                         ts_buf.at[pl.ds(j * 128, 128)], semt)
        return 0
    lax.fori_loop(0, 25, ts_f, 0)

    def m_body(j, _):
        gs = [pltpu.async_copy(
                  msg_hbm.at[ts_idx.at[pl.ds((j * 4 + q) * 128, 128)]],
                  rbufs[q], semg)
              for q in range(4)]
        ws = []
        for q in range(4):
            gs[q].wait()
            ws.append(pltpu.async_copy(
                rbufs[q],
                msg_out.at[pl.ds(lo + (j * 4 + q) * 128, 128), :], sems))
        for w in ws:
            w.wait()
        return 0
    lax.fori_loop(0, 6, m_body, 0)

    tg = pltpu.async_copy(
        msg_hbm.at[ts_idx.at[pl.ds(24 * 128, 64)]],
        rows_a.at[pl.ds(0, 64), :], semg)
    tg.wait()
    pltpu.async_copy(rows_a.at[pl.ds(0, 64), :],
                     msg_out.at[pl.ds(lo + 24 * 128, 64), :], sems).wait()

    nit = (ni + 127) // 128

    def z_f(j, _):
        pltpu.async_copy(zbuf, msg_out.at[idst2.at[j]], sems)
        return 0
    lax.fori_loop(0, nit, z_f, 0)

    def ts_d(j, _):
        pltpu.make_async_copy(ts_hbm.at[pl.ds(0, 128)],
                              ts_buf.at[pl.ds(j * 128, 128)], semt).wait()
        return 0
    lax.fori_loop(0, 25, ts_d, 0)

    def ts_m(g, _):
        v = vi_buf[pl.ds(g * 16, 16)].astype(jnp.float32)
        ts_buf[pl.ds(g * 16, 16)] = ts_buf[pl.ds(g * 16, 16)] * v
        return 0
    lax.fori_loop(0, GROUPS, ts_m, 0)
    pltpu.sync_copy(ts_buf.at[pl.ds(0, NT)], ts_out.at[pl.ds(lo, NT)])
    pltpu.sync_copy(vi_buf.at[pl.ds(0, NT)], vi_out.at[pl.ds(lo, NT)])

    def z_d(j, _):
        pltpu.make_async_copy(zbuf, msg_out.at[idst2.at[j]], sems).wait()
        return 0
    lax.fori_loop(0, nit, z_d, 0)


_k2 = pl.kernel(
    _body2,
    out_type=[
        jax.ShapeDtypeStruct((NPAD + 1, ND), jnp.float32),
        jax.ShapeDtypeStruct((NPAD,), jnp.float32),
        jax.ShapeDtypeStruct((NPAD,), jnp.int32),
    ],
    mesh=plsc.VectorSubcoreMesh(core_axis_name="c", subcore_axis_name="s"),
    compiler_params=pltpu.CompilerParams(needs_layout_passes=False),
    scratch_types=[
        pltpu.VMEM((NT,), jnp.int32),
        pltpu.VMEM((NT,), jnp.int32),
        pltpu.VMEM((NT,), jnp.int32),
        pltpu.VMEM((NT,), jnp.int32),
        pltpu.VMEM((NT,), jnp.int32),
        pltpu.VMEM((NT,), jnp.int32),
        pltpu.VMEM((NT,), jnp.int32),
        pltpu.VMEM((NT,), jnp.int32),
        pltpu.VMEM((NT,), jnp.int32),
        pltpu.VMEM((16,), jnp.int32),
        pltpu.VMEM((TBUF,), jnp.int32),
        pltpu.VMEM((TBUF,), jnp.int32),
        pltpu.VMEM((TBUF,), jnp.int32),
        pltpu.VMEM((NTR, 128), jnp.int32),
        pltpu.VMEM((TBUF,), jnp.float32),
        pltpu.VMEM((128, ND), jnp.float32),
        pltpu.VMEM((128, ND), jnp.float32),
        pltpu.VMEM((128, ND), jnp.float32),
        pltpu.VMEM((128, ND), jnp.float32),
        pltpu.VMEM((128, ND), jnp.float32),
        pltpu.SemaphoreType.DMA,
        pltpu.SemaphoreType.DMA,
        pltpu.SemaphoreType.DMA,
    ],
)


def kernel(node_ids, messages, timestamps, n_nodes):
    nid_pad = jnp.concatenate(
        [node_ids, jnp.zeros((NEPAD - NE,), jnp.int32)])
    (partials,) = _k1(nid_pad)
    zeros = jnp.zeros((128, ND), jnp.float32)
    nn = jnp.full((16,), n_nodes, jnp.int32)
    msg_o, ts_o, vi_o = _k2(partials, messages, timestamps, zeros, nn)
    return msg_o[:NNODES], ts_o[:NNODES], vi_o[:NNODES] > 0


# speedup vs baseline: 3.2479x; 3.2479x over previous
import jax
import jax.numpy as jnp
from jax import lax
from jax.experimental import pallas as pl
from jax.experimental.pallas import tpu as pltpu
from jax.experimental.pallas import tpu_sc as plsc

NE = 200000
ND = 128
NNODES = 100000
NW = 32
NT = 3136
NPAD = NW * NT
DUMP = NPAD
GROUPS = NT // 16
TBUF = 3328
NTR = TBUF // 128
ECH1 = 6256
NEPAD = NW * ECH1
EV1 = ECH1 // 16


def _shift_up(x, lane):
    idx = jnp.minimum(lane + 1, 15).reshape(16, 1)
    return lax.gather(
        x, idx,
        dimension_numbers=lax.GatherDimensionNumbers(
            offset_dims=(), collapsed_slice_dims=(0,), start_index_map=(0,)),
        slice_sizes=(1,),
        mode=lax.GatherScatterMode.PROMISE_IN_BOUNDS)


def _body1(nid_hbm, partials_out, lp_ref, ev_ref):
    wid = lax.axis_index("s") * 2 + lax.axis_index("c")
    lane = lax.iota(jnp.int32, 16)
    neg1 = jnp.full((16,), -1, jnp.int32)

    def init_lp(g, _):
        for q in range(4):
            lp_ref[pl.ds(g * 64 + q * 16, 16)] = neg1
        return 0
    lax.fori_loop(0, NPAD // 64, init_lp, 0)

    base = wid * ECH1
    pltpu.sync_copy(nid_hbm.at[pl.ds(base, ECH1)], ev_ref)

    def ev_vec(i, _):
        nid = ev_ref[pl.ds(i * 16, 16)]
        pos = base + i * 16 + lane
        key = lax.shift_left(nid, 4) + lane
        skey, spos = lax.sort([key, pos], num_keys=1)
        snid = lax.shift_right_arithmetic(skey, 4)
        nxt = _shift_up(snid, lane)
        is_last = (snid != nxt) | (lane == 15)
        m = is_last & (spos < NE)
        localc = jnp.clip(snid, 0, NPAD - 1)
        plsc.store_scatter(lp_ref, [localc], spos, mask=m)
        return 0
    lax.fori_loop(0, EV1, ev_vec, 0)

    pltpu.sync_copy(lp_ref, partials_out.at[pl.ds(wid * NPAD, NPAD)])


_k1 = pl.kernel(
    _body1,
    out_type=[jax.ShapeDtypeStruct((NW * NPAD,), jnp.int32)],
    mesh=plsc.VectorSubcoreMesh(core_axis_name="c", subcore_axis_name="s"),
    compiler_params=pltpu.CompilerParams(needs_layout_passes=False),
    scratch_types=[
        pltpu.VMEM((NPAD,), jnp.int32),
        pltpu.VMEM((ECH1,), jnp.int32),
    ],
)


def _body2(partials, msg_hbm, ts_hbm, zeros_hbm, nn_hbm,
           msg_out, ts_out, vi_out,
           lp_ref, pb0, pb1, pb2, pb3, pb4, pb5, pb6, pb7,
           nn_ref, ts_idx, vi_buf, idst, idst2, ts_buf,
           rows_a, rows_b, rows_c, rows_d, zbuf, semg, sems, semt):
    pbufs = [pb0, pb1, pb2, pb3, pb4, pb5, pb6, pb7]
    rbufs = [rows_a, rows_b, rows_c, rows_d]
    wid = lax.axis_index("s") * 2 + lax.axis_index("c")
    lo = wid * NT
    lane = lax.iota(jnp.int32, 16)

    pltpu.sync_copy(nn_hbm, nn_ref)
    pltpu.sync_copy(zeros_hbm, zbuf)

    neg1 = jnp.full((16,), -1, jnp.int32)
    zero16 = jnp.zeros((16,), jnp.int32)
    dump16 = jnp.full((16,), DUMP, jnp.int32)

    def init_lp(g, _):
        lp_ref[pl.ds(g * 16, 16)] = neg1
        return 0
    lax.fori_loop(0, GROUPS, init_lp, 0)

    def init_bufs(g, _):
        ts_idx[pl.ds(g * 16, 16)] = zero16
        idst[pl.ds(g * 16, 16)] = dump16
        return 0
    lax.fori_loop(0, TBUF // 16, init_bufs, 0)

    for b in range(4):
        das = [pltpu.async_copy(
                   partials.at[pl.ds((b * 8 + r) * NPAD + lo, NT)],
                   pbufs[r], semt)
               for r in range(8)]
        for d in das:
            d.wait()

        def mg(g, _):
            acc = lp_ref[pl.ds(g * 16, 16)]
            for r in range(8):
                acc = jnp.maximum(acc, pbufs[r][pl.ds(g * 16, 16)])
            lp_ref[pl.ds(g * 16, 16)] = acc
            return 0
        lax.fori_loop(0, GROUPS, mg, 0)

    nn = nn_ref[pl.ds(0, 16)]

    def a_body(g, ni):
        lp = lp_ref[pl.ds(g * 16, 16)]
        node = lo + g * 16 + lane
        valid = (lp >= 0) & (node < nn)
        safe = jnp.where(valid, lp, node)
        ts_idx[pl.ds(g * 16, 16)] = safe
        vi_buf[pl.ds(g * 16, 16)] = jnp.where(valid, 1, 0)
        inv = ~valid
        plsc.store_compressed(idst.at[pl.ds(ni, 16)], node, mask=inv)
        cv = jnp.max(plsc.all_reduce_population_count(valid))
        return ni + (16 - cv)

    ni = lax.fori_loop(0, GROUPS, a_body, jnp.int32(0))

    def c_body(j, _):
        for q in range(8):
            idst2[j, pl.ds(q * 16, 16)] = idst[pl.ds(j * 128 + q * 16, 16)]
        return 0
    lax.fori_loop(0, NTR, c_body, 0)

    def ts_f(j, _):
        pltpu.async_copy(ts_hbm.at[ts_idx.at[pl.ds(j * 128, 128)]],
                         ts_buf.at[pl.ds(j * 128, 128)], semt)
        return 0
    lax.fori_loop(0, 25, ts_f, 0)

    def m_body(j, _):
        gs = [pltpu.async_copy(
                  msg_hbm.at[ts_idx.at[pl.ds((j * 4 + q) * 128, 128)]],
                  rbufs[q], semg)
              for q in range(4)]
        ws = []
        for q in range(4):
            gs[q].wait()
            ws.append(pltpu.async_copy(
                rbufs[q],
                msg_out.at[pl.ds(lo + (j * 4 + q) * 128, 128), :], sems))
        for w in ws:
            w.wait()
        return 0
    lax.fori_loop(0, 6, m_body, 0)

    tg = pltpu.async_copy(
        msg_hbm.at[ts_idx.at[pl.ds(24 * 128, 64)]],
        rows_a.at[pl.ds(0, 64), :], semg)
    tg.wait()
    pltpu.async_copy(rows_a.at[pl.ds(0, 64), :],
                     msg_out.at[pl.ds(lo + 24 * 128, 64), :], sems).wait()

    nit = (ni + 127) // 128

    def z_f(j, _):
        pltpu.async_copy(zbuf, msg_out.at[idst2.at[j]], sems)
        return 0
    lax.fori_loop(0, nit, z_f, 0)

    def ts_d(j, _):
        pltpu.make_async_copy(ts_hbm.at[pl.ds(0, 128)],
                              ts_buf.at[pl.ds(j * 128, 128)], semt).wait()
        return 0
    lax.fori_loop(0, 25, ts_d, 0)

    def ts_m(g, _):
        v = vi_buf[pl.ds(g * 16, 16)].astype(jnp.float32)
        ts_buf[pl.ds(g * 16, 16)] = ts_buf[pl.ds(g * 16, 16)] * v
        return 0
    lax.fori_loop(0, GROUPS, ts_m, 0)
    pltpu.sync_copy(ts_buf.at[pl.ds(0, NT)], ts_out.at[pl.ds(lo, NT)])
    pltpu.sync_copy(vi_buf.at[pl.ds(0, NT)], vi_out.at[pl.ds(lo, NT)])

    def z_d(j, _):
        pltpu.make_async_copy(zbuf, msg_out.at[idst2.at[j]], sems).wait()
        return 0
    lax.fori_loop(0, nit, z_d, 0)


_k2 = pl.kernel(
    _body2,
    out_type=[
        jax.ShapeDtypeStruct((NPAD + 1, ND), jnp.float32),
        jax.ShapeDtypeStruct((NPAD,), jnp.float32),
        jax.ShapeDtypeStruct((NPAD,), jnp.int32),
    ],
    mesh=plsc.VectorSubcoreMesh(core_axis_name="c", subcore_axis_name="s"),
    compiler_params=pltpu.CompilerParams(needs_layout_passes=False),
    scratch_types=[
        pltpu.VMEM((NT,), jnp.int32),
        pltpu.VMEM((NT,), jnp.int32),
        pltpu.VMEM((NT,), jnp.int32),
        pltpu.VMEM((NT,), jnp.int32),
        pltpu.VMEM((NT,), jnp.int32),
        pltpu.VMEM((NT,), jnp.int32),
        pltpu.VMEM((NT,), jnp.int32),
        pltpu.VMEM((NT,), jnp.int32),
        pltpu.VMEM((NT,), jnp.int32),
        pltpu.VMEM((16,), jnp.int32),
        pltpu.VMEM((TBUF,), jnp.int32),
        pltpu.VMEM((TBUF,), jnp.int32),
        pltpu.VMEM((TBUF,), jnp.int32),
        pltpu.VMEM((NTR, 128), jnp.int32),
        pltpu.VMEM((TBUF,), jnp.float32),
        pltpu.VMEM((128, ND), jnp.float32),
        pltpu.VMEM((128, ND), jnp.float32),
        pltpu.VMEM((128, ND), jnp.float32),
        pltpu.VMEM((128, ND), jnp.float32),
        pltpu.VMEM((128, ND), jnp.float32),
        pltpu.SemaphoreType.DMA,
        pltpu.SemaphoreType.DMA,
        pltpu.SemaphoreType.DMA,
    ],
)


def kernel(node_ids, messages, timestamps, n_nodes):
    nid_pad = jnp.concatenate(
        [node_ids, jnp.zeros((NEPAD - NE,), jnp.int32)])
    (partials,) = _k1(nid_pad)
    zeros = jnp.zeros((128, ND), jnp.float32)
    nn = jnp.full((16,), n_nodes, jnp.int32)
    msg_o, ts_o, vi_o = _k2(partials, messages, timestamps, zeros, nn)
    return msg_o[:NNODES], ts_o[:NNODES], vi_o[:NNODES] > 0


# speedup vs baseline: 3.2630x; 1.0046x over previous
import jax
import jax.numpy as jnp
from jax import lax
from jax.experimental import pallas as pl
from jax.experimental.pallas import tpu as pltpu
from jax.experimental.pallas import tpu_sc as plsc

NE = 200000
ND = 128
NNODES = 100000
NW = 32
NT = 3136
NPAD = NW * NT
DUMP = NPAD
GROUPS = NT // 16
TBUF = 3328
NTR = TBUF // 128
ECH1 = 6256
NEPAD = NW * ECH1
EV1 = ECH1 // 16


def _shift_up(x, lane):
    idx = jnp.minimum(lane + 1, 15).reshape(16, 1)
    return lax.gather(
        x, idx,
        dimension_numbers=lax.GatherDimensionNumbers(
            offset_dims=(), collapsed_slice_dims=(0,), start_index_map=(0,)),
        slice_sizes=(1,),
        mode=lax.GatherScatterMode.PROMISE_IN_BOUNDS)


def _body1(nid_hbm, partials_out, lp_ref, ev_ref):
    wid = lax.axis_index("s") * 2 + lax.axis_index("c")
    lane = lax.iota(jnp.int32, 16)
    neg1 = jnp.full((16,), -1, jnp.int32)

    def init_lp(g, _):
        for q in range(4):
            lp_ref[pl.ds(g * 64 + q * 16, 16)] = neg1
        return 0
    lax.fori_loop(0, NPAD // 64, init_lp, 0)

    raw = wid * (NE // NW)
    base = pl.multiple_of(raw - lax.rem(raw, 8), 8)
    pltpu.sync_copy(nid_hbm.at[pl.ds(base, ECH1)], ev_ref)

    def ev_vec(i, _):
        nid = ev_ref[pl.ds(i * 16, 16)]
        pos = base + i * 16 + lane
        key = lax.shift_left(nid, 4) + lane
        skey, spos = lax.sort([key, pos], num_keys=1)
        snid = lax.shift_right_arithmetic(skey, 4)
        nxt = _shift_up(snid, lane)
        m = (snid != nxt) | (lane == 15)
        localc = jnp.clip(snid, 0, NPAD - 1)
        plsc.store_scatter(lp_ref, [localc], spos, mask=m)
        return 0
    lax.fori_loop(0, EV1, ev_vec, 0)

    pltpu.sync_copy(lp_ref, partials_out.at[pl.ds(wid * NPAD, NPAD)])


_k1 = pl.kernel(
    _body1,
    out_type=[jax.ShapeDtypeStruct((NW * NPAD,), jnp.int32)],
    mesh=plsc.VectorSubcoreMesh(core_axis_name="c", subcore_axis_name="s"),
    compiler_params=pltpu.CompilerParams(needs_layout_passes=False),
    scratch_types=[
        pltpu.VMEM((NPAD,), jnp.int32),
        pltpu.VMEM((ECH1,), jnp.int32),
    ],
)


def _body2(partials, msg_hbm, ts_hbm, zeros_hbm, nn_hbm,
           msg_out, ts_out, vi_out,
           lp_ref, pb0, pb1, pb2, pb3, pb4, pb5, pb6, pb7,
           nn_ref, ts_idx, vi_buf, idst, idst2, ts_buf,
           rows_a, rows_b, rows_c, rows_d, zbuf, semg, sems, semt,
           semw0, semw1, semw2, semw3):
    pbufs = [pb0, pb1, pb2, pb3, pb4, pb5, pb6, pb7]
    rbufs = [rows_a, rows_b, rows_c, rows_d]
    semw = [semw0, semw1, semw2, semw3]
    wid = lax.axis_index("s") * 2 + lax.axis_index("c")
    lo = wid * NT
    lane = lax.iota(jnp.int32, 16)

    pltpu.sync_copy(nn_hbm, nn_ref)
    pltpu.sync_copy(zeros_hbm, zbuf)

    neg1 = jnp.full((16,), -1, jnp.int32)
    zero16 = jnp.zeros((16,), jnp.int32)
    dump16 = jnp.full((16,), DUMP, jnp.int32)

    def init_lp(g, _):
        lp_ref[pl.ds(g * 16, 16)] = neg1
        return 0
    lax.fori_loop(0, GROUPS, init_lp, 0)

    def init_bufs(g, _):
        ts_idx[pl.ds(g * 16, 16)] = zero16
        idst[pl.ds(g * 16, 16)] = dump16
        return 0
    lax.fori_loop(0, TBUF // 16, init_bufs, 0)

    for b in range(4):
        das = [pltpu.async_copy(
                   partials.at[pl.ds((b * 8 + r) * NPAD + lo, NT)],
                   pbufs[r], semt)
               for r in range(8)]
        for d in das:
            d.wait()

        def mg(g, _):
            acc = lp_ref[pl.ds(g * 16, 16)]
            for r in range(8):
                acc = jnp.maximum(acc, pbufs[r][pl.ds(g * 16, 16)])
            lp_ref[pl.ds(g * 16, 16)] = acc
            return 0
        lax.fori_loop(0, GROUPS, mg, 0)

    nn = nn_ref[pl.ds(0, 16)]

    def a_body(g, ni):
        lp = lp_ref[pl.ds(g * 16, 16)]
        node = lo + g * 16 + lane
        valid = (lp >= 0) & (node < nn)
        safe = jnp.where(valid, lp, node)
        ts_idx[pl.ds(g * 16, 16)] = safe
        vi_buf[pl.ds(g * 16, 16)] = jnp.where(valid, 1, 0)
        inv = ~valid
        plsc.store_compressed(idst.at[pl.ds(ni, 16)], node, mask=inv)
        cv = jnp.max(plsc.all_reduce_population_count(valid))
        return ni + (16 - cv)

    ni = lax.fori_loop(0, GROUPS, a_body, jnp.int32(0))

    def c_body(j, _):
        for q in range(8):
            idst2[j, pl.ds(q * 16, 16)] = idst[pl.ds(j * 128 + q * 16, 16)]
        return 0
    lax.fori_loop(0, NTR, c_body, 0)

    def ts_f(j, _):
        pltpu.async_copy(ts_hbm.at[ts_idx.at[pl.ds(j * 128, 128)]],
                         ts_buf.at[pl.ds(j * 128, 128)], semt)
        return 0
    lax.fori_loop(0, 25, ts_f, 0)

    def m_body(j, _):
        for q in range(4):

            @pl.when(j > 0)
            def _():
                pltpu.make_async_copy(
                    rbufs[q],
                    msg_out.at[pl.ds(lo + (j * 4 + q - 4) * 128, 128), :],
                    semw[q]).wait()

            pltpu.async_copy(
                msg_hbm.at[ts_idx.at[pl.ds((j * 4 + q) * 128, 128)]],
                rbufs[q], semg)
        for q in range(4):
            pltpu.make_async_copy(
                msg_hbm.at[pl.ds(0, 128), :], rbufs[q], semg).wait()
            pltpu.async_copy(
                rbufs[q],
                msg_out.at[pl.ds(lo + (j * 4 + q) * 128, 128), :], semw[q])
        return 0
    lax.fori_loop(0, 6, m_body, 0)

    for q in range(4):
        pltpu.make_async_copy(
            rbufs[q], msg_out.at[pl.ds(lo + (20 + q) * 128, 128), :],
            semw[q]).wait()

    tg = pltpu.async_copy(
        msg_hbm.at[ts_idx.at[pl.ds(24 * 128, 64)]],
        rows_a.at[pl.ds(0, 64), :], semg)
    tg.wait()
    pltpu.async_copy(rows_a.at[pl.ds(0, 64), :],
                     msg_out.at[pl.ds(lo + 24 * 128, 64), :], sems).wait()

    nit = (ni + 127) // 128

    def z_f(j, _):
        pltpu.async_copy(zbuf, msg_out.at[idst2.at[j]], sems)
        return 0
    lax.fori_loop(0, nit, z_f, 0)

    def ts_d(j, _):
        pltpu.make_async_copy(ts_hbm.at[pl.ds(0, 128)],
                              ts_buf.at[pl.ds(j * 128, 128)], semt).wait()
        return 0
    lax.fori_loop(0, 25, ts_d, 0)

    def ts_m(g, _):
        v = vi_buf[pl.ds(g * 16, 16)].astype(jnp.float32)
        ts_buf[pl.ds(g * 16, 16)] = ts_buf[pl.ds(g * 16, 16)] * v
        return 0
    lax.fori_loop(0, GROUPS, ts_m, 0)
    pltpu.sync_copy(ts_buf.at[pl.ds(0, NT)], ts_out.at[pl.ds(lo, NT)])
    pltpu.sync_copy(vi_buf.at[pl.ds(0, NT)], vi_out.at[pl.ds(lo, NT)])

    def z_d(j, _):
        pltpu.make_async_copy(zbuf, msg_out.at[idst2.at[j]], sems).wait()
        return 0
    lax.fori_loop(0, nit, z_d, 0)


_k2 = pl.kernel(
    _body2,
    out_type=[
        jax.ShapeDtypeStruct((NPAD + 1, ND), jnp.float32),
        jax.ShapeDtypeStruct((NPAD,), jnp.float32),
        jax.ShapeDtypeStruct((NPAD,), jnp.int32),
    ],
    mesh=plsc.VectorSubcoreMesh(core_axis_name="c", subcore_axis_name="s"),
    compiler_params=pltpu.CompilerParams(needs_layout_passes=False),
    scratch_types=[
        pltpu.VMEM((NT,), jnp.int32),
        pltpu.VMEM((NT,), jnp.int32),
        pltpu.VMEM((NT,), jnp.int32),
        pltpu.VMEM((NT,), jnp.int32),
        pltpu.VMEM((NT,), jnp.int32),
        pltpu.VMEM((NT,), jnp.int32),
        pltpu.VMEM((NT,), jnp.int32),
        pltpu.VMEM((NT,), jnp.int32),
        pltpu.VMEM((NT,), jnp.int32),
        pltpu.VMEM((16,), jnp.int32),
        pltpu.VMEM((TBUF,), jnp.int32),
        pltpu.VMEM((TBUF,), jnp.int32),
        pltpu.VMEM((TBUF,), jnp.int32),
        pltpu.VMEM((NTR, 128), jnp.int32),
        pltpu.VMEM((TBUF,), jnp.float32),
        pltpu.VMEM((128, ND), jnp.float32),
        pltpu.VMEM((128, ND), jnp.float32),
        pltpu.VMEM((128, ND), jnp.float32),
        pltpu.VMEM((128, ND), jnp.float32),
        pltpu.VMEM((128, ND), jnp.float32),
        pltpu.SemaphoreType.DMA,
        pltpu.SemaphoreType.DMA,
        pltpu.SemaphoreType.DMA,
        pltpu.SemaphoreType.DMA,
        pltpu.SemaphoreType.DMA,
        pltpu.SemaphoreType.DMA,
        pltpu.SemaphoreType.DMA,
    ],
)


def kernel(node_ids, messages, timestamps, n_nodes):
    (partials,) = _k1(node_ids)
    zeros = jnp.zeros((128, ND), jnp.float32)
    nn = jnp.full((16,), n_nodes, jnp.int32)
    msg_o, ts_o, vi_o = _k2(partials, messages, timestamps, zeros, nn)
    return msg_o[:NNODES], ts_o[:NNODES], vi_o[:NNODES] > 0


# speedup vs baseline: 6.5424x; 2.0051x over previous
import jax
import jax.numpy as jnp
from jax import lax
from jax.experimental import pallas as pl
from jax.experimental.pallas import tpu as pltpu
from jax.experimental.pallas import tpu_sc as plsc

NE = 200000
ND = 128
NNODES = 100000
NW = 32
NT = 3136
NPAD = NW * NT
DUMP = NPAD
GROUPS = NT // 16
TBUF = 3328
NTR = TBUF // 128
ECH1 = 6256
NEPAD = NW * ECH1
EV1 = ECH1 // 16


def _shift_up(x, lane):
    idx = jnp.minimum(lane + 1, 15).reshape(16, 1)
    return lax.gather(
        x, idx,
        dimension_numbers=lax.GatherDimensionNumbers(
            offset_dims=(), collapsed_slice_dims=(0,), start_index_map=(0,)),
        slice_sizes=(1,),
        mode=lax.GatherScatterMode.PROMISE_IN_BOUNDS)


def _body1(nid_hbm, partials_out, lp_ref, ev_ref):
    wid = lax.axis_index("s") * 2 + lax.axis_index("c")
    lane = lax.iota(jnp.int32, 16)
    neg1 = jnp.full((16,), -1, jnp.int32)

    def init_lp(g, _):
        for q in range(4):
            lp_ref[pl.ds(g * 64 + q * 16, 16)] = neg1
        return 0
    lax.fori_loop(0, NPAD // 64, init_lp, 0)

    raw = wid * (NE // NW)
    base = pl.multiple_of(raw - lax.rem(raw, 8), 8)
    pltpu.sync_copy(nid_hbm.at[pl.ds(base, ECH1)], ev_ref)

    def ev_vec(i, _):
        nid = ev_ref[pl.ds(i * 16, 16)]
        pos = base + i * 16 + lane
        key = lax.shift_left(nid, 4) + lane
        skey, spos = lax.sort([key, pos], num_keys=1)
        snid = lax.shift_right_arithmetic(skey, 4)
        nxt = _shift_up(snid, lane)
        m = (snid != nxt) | (lane == 15)
        localc = jnp.clip(snid, 0, NPAD - 1)
        plsc.store_scatter(lp_ref, [localc], spos, mask=m)
        return 0
    lax.fori_loop(0, EV1, ev_vec, 0)

    pltpu.sync_copy(lp_ref, partials_out.at[pl.ds(wid * NPAD, NPAD)])


_k1 = pl.kernel(
    _body1,
    out_type=[jax.ShapeDtypeStruct((NW * NPAD,), jnp.int32)],
    mesh=plsc.VectorSubcoreMesh(core_axis_name="c", subcore_axis_name="s"),
    compiler_params=pltpu.CompilerParams(needs_layout_passes=False),
    scratch_types=[
        pltpu.VMEM((NPAD,), jnp.int32),
        pltpu.VMEM((ECH1,), jnp.int32),
    ],
)


def _body2(partials, msg_hbm, ts_hbm, zeros_hbm, nn_hbm,
           msg_out, ts_out, vi_out,
           lp_ref, pb0, pb1, pb2, pb3, pb4, pb5, pb6, pb7,
           nn_ref, ts_idx, vi_buf, idst, idst2, ts_buf,
           rows_a, rows_b, rows_c, rows_d, zbuf, semg, sems, semt,
           semw0, semw1, semw2, semw3):
    pbufs = [pb0, pb1, pb2, pb3, pb4, pb5, pb6, pb7]
    rbufs = [rows_a, rows_b, rows_c, rows_d]
    semw = [semw0, semw1, semw2, semw3]
    wid = lax.axis_index("s") * 2 + lax.axis_index("c")
    lo = wid * NT
    lane = lax.iota(jnp.int32, 16)

    pltpu.sync_copy(nn_hbm, nn_ref)
    pltpu.sync_copy(zeros_hbm, zbuf)

    neg1 = jnp.full((16,), -1, jnp.int32)
    zero16 = jnp.zeros((16,), jnp.int32)
    dump16 = jnp.full((16,), DUMP, jnp.int32)

    def init_lp(g, _):
        lp_ref[pl.ds(g * 16, 16)] = neg1
        return 0
    lax.fori_loop(0, GROUPS, init_lp, 0)

    def init_bufs(g, _):
        ts_idx[pl.ds(g * 16, 16)] = zero16
        idst[pl.ds(g * 16, 16)] = dump16
        return 0
    lax.fori_loop(0, TBUF // 16, init_bufs, 0)

    for b in range(4):
        das = [pltpu.async_copy(
                   partials.at[pl.ds((b * 8 + r) * NPAD + lo, NT)],
                   pbufs[r], semt)
               for r in range(8)]
        for d in das:
            d.wait()

        def mg(g, _):
            acc = lp_ref[pl.ds(g * 16, 16)]
            for r in range(8):
                acc = jnp.maximum(acc, pbufs[r][pl.ds(g * 16, 16)])
            lp_ref[pl.ds(g * 16, 16)] = acc
            return 0
        lax.fori_loop(0, GROUPS, mg, 0)

    nn = nn_ref[pl.ds(0, 16)]

    def a_body(g, ni):
        lp = lp_ref[pl.ds(g * 16, 16)]
        node = lo + g * 16 + lane
        valid = (lp >= 0) & (node < nn)
        safe = jnp.where(valid, lp, node)
        ts_idx[pl.ds(g * 16, 16)] = safe
        vi_buf[pl.ds(g * 16, 16)] = jnp.where(valid, 1, 0)
        inv = (~valid) & (node < NNODES)
        plsc.store_compressed(idst.at[pl.ds(ni, 16)], node, mask=inv)
        cv = jnp.max(plsc.all_reduce_population_count(inv))
        return ni + cv

    ni = lax.fori_loop(0, GROUPS, a_body, jnp.int32(0))

    lastn = plsc.load_gather(idst, [jnp.full((16,), jnp.maximum(ni - 1, 0),
                                             jnp.int32)])
    for k in range(8):
        idst[pl.ds(ni + k * 16, 16)] = lastn

    def c_body(j, _):
        for q in range(8):
            idst2[j, pl.ds(q * 16, 16)] = idst[pl.ds(j * 128 + q * 16, 16)]
        return 0
    lax.fori_loop(0, NTR, c_body, 0)

    def ts_f(j, _):
        pltpu.async_copy(ts_hbm.at[ts_idx.at[pl.ds(j * 128, 128)]],
                         ts_buf.at[pl.ds(j * 128, 128)], semt)
        return 0
    lax.fori_loop(0, 25, ts_f, 0)

    def _mw(c, q, drain):
        start = lo + c * 128

        @pl.when(start + 128 <= NNODES)
        def _():
            d = pltpu.make_async_copy(
                rbufs[q], msg_out.at[pl.ds(start, 128), :], semw[q])
            if drain:
                d.wait()
            else:
                d.start()

        @pl.when((start < NNODES) & (start + 128 > NNODES))
        def _():
            d = pltpu.make_async_copy(
                rbufs[q].at[pl.ds(0, 96), :],
                msg_out.at[pl.ds(start, 96), :], semw[q])
            if drain:
                d.wait()
            else:
                d.start()

    def m_body(j, _):
        for q in range(4):

            @pl.when(j > 0)
            def _():
                _mw(j * 4 + q - 4, q, drain=True)

            pltpu.async_copy(
                msg_hbm.at[ts_idx.at[pl.ds((j * 4 + q) * 128, 128)]],
                rbufs[q], semg)
        for q in range(4):
            pltpu.make_async_copy(
                msg_hbm.at[pl.ds(0, 128), :], rbufs[q], semg).wait()
            _mw(j * 4 + q, q, drain=False)
        return 0
    lax.fori_loop(0, 6, m_body, 0)

    for q in range(4):
        _mw(20 + q, q, drain=True)

    @pl.when(lo + 24 * 128 + 64 <= NNODES)
    def _():
        tg = pltpu.async_copy(
            msg_hbm.at[ts_idx.at[pl.ds(24 * 128, 64)]],
            rows_a.at[pl.ds(0, 64), :], semg)
        tg.wait()
        pltpu.async_copy(rows_a.at[pl.ds(0, 64), :],
                         msg_out.at[pl.ds(lo + 24 * 128, 64), :], sems).wait()

    nit = (ni + 127) // 128

    def z_f(j, _):
        pltpu.async_copy(zbuf, msg_out.at[idst2.at[j]], sems)
        return 0
    lax.fori_loop(0, nit, z_f, 0)

    def ts_d(j, _):
        pltpu.make_async_copy(ts_hbm.at[pl.ds(0, 128)],
                              ts_buf.at[pl.ds(j * 128, 128)], semt).wait()
        return 0
    lax.fori_loop(0, 25, ts_d, 0)

    def ts_m(g, _):
        v = vi_buf[pl.ds(g * 16, 16)].astype(jnp.float32)
        ts_buf[pl.ds(g * 16, 16)] = ts_buf[pl.ds(g * 16, 16)] * v
        return 0
    lax.fori_loop(0, GROUPS, ts_m, 0)
    NLAST = NNODES - (NW - 1) * NT

    @pl.when(lo + NT <= NNODES)
    def _():
        pltpu.sync_copy(ts_buf.at[pl.ds(0, NT)], ts_out.at[pl.ds(lo, NT)])
        pltpu.sync_copy(vi_buf.at[pl.ds(0, NT)], vi_out.at[pl.ds(lo, NT)])

    @pl.when(lo + NT > NNODES)
    def _():
        pltpu.sync_copy(ts_buf.at[pl.ds(0, NLAST)],
                        ts_out.at[pl.ds(lo, NLAST)])
        pltpu.sync_copy(vi_buf.at[pl.ds(0, NLAST)],
                        vi_out.at[pl.ds(lo, NLAST)])

    def z_d(j, _):
        pltpu.make_async_copy(zbuf, msg_out.at[idst2.at[j]], sems).wait()
        return 0
    lax.fori_loop(0, nit, z_d, 0)


_k2 = pl.kernel(
    _body2,
    out_type=[
        jax.ShapeDtypeStruct((NNODES, ND), jnp.float32),
        jax.ShapeDtypeStruct((NNODES,), jnp.float32),
        jax.ShapeDtypeStruct((NNODES,), jnp.int32),
    ],
    mesh=plsc.VectorSubcoreMesh(core_axis_name="c", subcore_axis_name="s"),
    compiler_params=pltpu.CompilerParams(needs_layout_passes=False),
    scratch_types=[
        pltpu.VMEM((NT,), jnp.int32),
        pltpu.VMEM((NT,), jnp.int32),
        pltpu.VMEM((NT,), jnp.int32),
        pltpu.VMEM((NT,), jnp.int32),
        pltpu.VMEM((NT,), jnp.int32),
        pltpu.VMEM((NT,), jnp.int32),
        pltpu.VMEM((NT,), jnp.int32),
        pltpu.VMEM((NT,), jnp.int32),
        pltpu.VMEM((NT,), jnp.int32),
        pltpu.VMEM((16,), jnp.int32),
        pltpu.VMEM((TBUF,), jnp.int32),
        pltpu.VMEM((TBUF,), jnp.int32),
        pltpu.VMEM((TBUF,), jnp.int32),
        pltpu.VMEM((NTR, 128), jnp.int32),
        pltpu.VMEM((TBUF,), jnp.float32),
        pltpu.VMEM((128, ND), jnp.float32),
        pltpu.VMEM((128, ND), jnp.float32),
        pltpu.VMEM((128, ND), jnp.float32),
        pltpu.VMEM((128, ND), jnp.float32),
        pltpu.VMEM((128, ND), jnp.float32),
        pltpu.SemaphoreType.DMA,
        pltpu.SemaphoreType.DMA,
        pltpu.SemaphoreType.DMA,
        pltpu.SemaphoreType.DMA,
        pltpu.SemaphoreType.DMA,
        pltpu.SemaphoreType.DMA,
        pltpu.SemaphoreType.DMA,
    ],
)


def kernel(node_ids, messages, timestamps, n_nodes):
    (partials,) = _k1(node_ids)
    zeros = jnp.zeros((128, ND), jnp.float32)
    nn = jnp.full((16,), n_nodes, jnp.int32)
    msg_o, ts_o, vi_o = _k2(partials, messages, timestamps, zeros, nn)
    return msg_o, ts_o, vi_o > 0


# speedup vs baseline: 6.8356x; 1.0448x over previous
import jax
import jax.numpy as jnp
from jax import lax
from jax.experimental import pallas as pl
from jax.experimental.pallas import tpu as pltpu
from jax.experimental.pallas import tpu_sc as plsc

NE = 200000
ND = 128
NNODES = 100000
NW = 32
NT = 3136
NPAD = NW * NT
DUMP = NPAD
GROUPS = NT // 16
TBUF = 3328
NTR = TBUF // 128
ECH1 = 6256
NEPAD = NW * ECH1
EV1 = ECH1 // 16


def _shift_up(x, lane):
    idx = jnp.minimum(lane + 1, 15).reshape(16, 1)
    return lax.gather(
        x, idx,
        dimension_numbers=lax.GatherDimensionNumbers(
            offset_dims=(), collapsed_slice_dims=(0,), start_index_map=(0,)),
        slice_sizes=(1,),
        mode=lax.GatherScatterMode.PROMISE_IN_BOUNDS)


def _body1(nid_hbm, partials_out, lp_ref, ev_ref):
    wid = lax.axis_index("s") * 2 + lax.axis_index("c")
    lane = lax.iota(jnp.int32, 16)
    neg1 = jnp.full((16,), -1, jnp.int32)

    def init_lp(g, _):
        for q in range(4):
            lp_ref[pl.ds(g * 64 + q * 16, 16)] = neg1
        return 0
    lax.fori_loop(0, NPAD // 64, init_lp, 0)

    raw = wid * (NE // NW)
    base = pl.multiple_of(raw - lax.rem(raw, 8), 8)
    pltpu.sync_copy(nid_hbm.at[pl.ds(base, ECH1)], ev_ref)

    def ev_vec(i, _):
        nid = ev_ref[pl.ds(i * 16, 16)]
        pos = base + i * 16 + lane
        key = lax.shift_left(nid, 4) + lane
        skey, spos = lax.sort([key, pos], num_keys=1)
        snid = lax.shift_right_arithmetic(skey, 4)
        nxt = _shift_up(snid, lane)
        m = (snid != nxt) | (lane == 15)
        localc = jnp.clip(snid, 0, NPAD - 1)
        plsc.store_scatter(lp_ref, [localc], spos, mask=m)
        return 0
    lax.fori_loop(0, EV1, ev_vec, 0)

    pltpu.sync_copy(lp_ref, partials_out.at[pl.ds(wid * NPAD, NPAD)])


_k1 = pl.kernel(
    _body1,
    out_type=[jax.ShapeDtypeStruct((NW * NPAD,), jnp.int32)],
    mesh=plsc.VectorSubcoreMesh(core_axis_name="c", subcore_axis_name="s"),
    compiler_params=pltpu.CompilerParams(needs_layout_passes=False),
    scratch_types=[
        pltpu.VMEM((NPAD,), jnp.int32),
        pltpu.VMEM((ECH1,), jnp.int32),
    ],
)


def _body2(partials, msg_hbm, ts_hbm, zeros_hbm, nn_hbm,
           msg_out, ts_out, vi_out,
           lp_ref, pb0, pb1, pb2, pb3, pb4, pb5, pb6, pb7,
           nn_ref, ts_idx, vi_buf, idst, idst2, ts_buf,
           rows_a, rows_b, rows_c, rows_d, zbuf, semg, sems, semt,
           semw0, semw1, semw2, semw3, semm):
    pbufs = [pb0, pb1, pb2, pb3, pb4, pb5, pb6, pb7]
    rbufs = [rows_a, rows_b, rows_c, rows_d]
    semw = [semw0, semw1, semw2, semw3]
    wid = lax.axis_index("s") * 2 + lax.axis_index("c")
    lo = wid * NT
    lane = lax.iota(jnp.int32, 16)

    pltpu.sync_copy(nn_hbm, nn_ref)
    pltpu.sync_copy(zeros_hbm, zbuf)

    neg1 = jnp.full((16,), -1, jnp.int32)
    zero16 = jnp.zeros((16,), jnp.int32)
    dump16 = jnp.full((16,), DUMP, jnp.int32)

    def init_lp(g, _):
        lp_ref[pl.ds(g * 16, 16)] = neg1
        return 0
    lax.fori_loop(0, GROUPS, init_lp, 0)

    def init_bufs(g, _):
        ts_idx[pl.ds(g * 16, 16)] = zero16
        idst[pl.ds(g * 16, 16)] = dump16
        return 0
    lax.fori_loop(0, TBUF // 16, init_bufs, 0)

    def _fire(b, bufs):
        return [pltpu.async_copy(
                    partials.at[pl.ds((b * 4 + r) * NPAD + lo, NT)],
                    bufs[r], semt if b % 2 == 0 else semm)
                for r in range(4)]

    _fire(0, pbufs[:4])
    for b in range(8):
        cur = pbufs[:4] if b % 2 == 0 else pbufs[4:]
        nxt = pbufs[4:] if b % 2 == 0 else pbufs[:4]
        for r in range(4):
            pltpu.make_async_copy(
                partials.at[pl.ds((b * 4 + r) * NPAD + lo, NT)],
                cur[r], semt if b % 2 == 0 else semm).wait()
        if b < 7:
            _fire(b + 1, nxt)

        def mg(g, _):
            acc = lp_ref[pl.ds(g * 16, 16)]
            for r in range(4):
                acc = jnp.maximum(acc, cur[r][pl.ds(g * 16, 16)])
            lp_ref[pl.ds(g * 16, 16)] = acc
            return 0
        lax.fori_loop(0, GROUPS, mg, 0)

    nn = nn_ref[pl.ds(0, 16)]

    def a_body(g, ni):
        lp = lp_ref[pl.ds(g * 16, 16)]
        node = lo + g * 16 + lane
        valid = (lp >= 0) & (node < nn)
        safe = jnp.where(valid, lp, node)
        ts_idx[pl.ds(g * 16, 16)] = safe
        vi_buf[pl.ds(g * 16, 16)] = jnp.where(valid, 1, 0)
        inv = (~valid) & (node < NNODES)
        plsc.store_compressed(idst.at[pl.ds(ni, 16)], node, mask=inv)
        cv = jnp.max(plsc.all_reduce_population_count(inv))
        return ni + cv

    ni = lax.fori_loop(0, 32, a_body, jnp.int32(0))
    for q in range(4):
        pltpu.async_copy(
            msg_hbm.at[ts_idx.at[pl.ds(q * 128, 128)]], rbufs[q], semg)
    ni = lax.fori_loop(32, GROUPS, a_body, ni)

    lastn = plsc.load_gather(idst, [jnp.full((16,), jnp.maximum(ni - 1, 0),
                                             jnp.int32)])
    for k in range(8):
        idst[pl.ds(ni + k * 16, 16)] = lastn

    def c_body(j, _):
        for q in range(8):
            idst2[j, pl.ds(q * 16, 16)] = idst[pl.ds(j * 128 + q * 16, 16)]
        return 0
    lax.fori_loop(0, NTR, c_body, 0)

    def ts_f(j, _):
        pltpu.async_copy(ts_hbm.at[ts_idx.at[pl.ds(j * 128, 128)]],
                         ts_buf.at[pl.ds(j * 128, 128)], semt)
        return 0
    lax.fori_loop(0, 25, ts_f, 0)

    def _mw(c, q, drain):
        start = lo + c * 128

        @pl.when(start + 128 <= NNODES)
        def _():
            d = pltpu.make_async_copy(
                rbufs[q], msg_out.at[pl.ds(start, 128), :], semw[q])
            if drain:
                d.wait()
            else:
                d.start()

        @pl.when((start < NNODES) & (start + 128 > NNODES))
        def _():
            d = pltpu.make_async_copy(
                rbufs[q].at[pl.ds(0, 96), :],
                msg_out.at[pl.ds(start, 96), :], semw[q])
            if drain:
                d.wait()
            else:
                d.start()

    def m_body(j, _):
        for q in range(4):

            @pl.when(j > 0)
            def _():
                _mw(j * 4 + q - 4, q, drain=True)
                pltpu.async_copy(
                    msg_hbm.at[ts_idx.at[pl.ds((j * 4 + q) * 128, 128)]],
                    rbufs[q], semg)
        for q in range(4):
            pltpu.make_async_copy(
                msg_hbm.at[pl.ds(0, 128), :], rbufs[q], semg).wait()
            _mw(j * 4 + q, q, drain=False)
        return 0
    lax.fori_loop(0, 6, m_body, 0)

    for q in range(4):
        _mw(20 + q, q, drain=True)

    @pl.when(lo + 24 * 128 + 64 <= NNODES)
    def _():
        tg = pltpu.async_copy(
            msg_hbm.at[ts_idx.at[pl.ds(24 * 128, 64)]],
            rows_a.at[pl.ds(0, 64), :], semg)
        tg.wait()
        pltpu.async_copy(rows_a.at[pl.ds(0, 64), :],
                         msg_out.at[pl.ds(lo + 24 * 128, 64), :], sems).wait()

    nit = (ni + 127) // 128

    def z_f(j, _):
        pltpu.async_copy(zbuf, msg_out.at[idst2.at[j]], sems)
        return 0
    lax.fori_loop(0, nit, z_f, 0)

    def ts_d(j, _):
        pltpu.make_async_copy(ts_hbm.at[pl.ds(0, 128)],
                              ts_buf.at[pl.ds(j * 128, 128)], semt).wait()
        return 0
    lax.fori_loop(0, 25, ts_d, 0)

    def ts_m(g, _):
        v = vi_buf[pl.ds(g * 16, 16)].astype(jnp.float32)
        ts_buf[pl.ds(g * 16, 16)] = ts_buf[pl.ds(g * 16, 16)] * v
        return 0
    lax.fori_loop(0, GROUPS, ts_m, 0)
    NLAST = NNODES - (NW - 1) * NT

    @pl.when(lo + NT <= NNODES)
    def _():
        pltpu.sync_copy(ts_buf.at[pl.ds(0, NT)], ts_out.at[pl.ds(lo, NT)])
        pltpu.sync_copy(vi_buf.at[pl.ds(0, NT)], vi_out.at[pl.ds(lo, NT)])

    @pl.when(lo + NT > NNODES)
    def _():
        pltpu.sync_copy(ts_buf.at[pl.ds(0, NLAST)],
                        ts_out.at[pl.ds(lo, NLAST)])
        pltpu.sync_copy(vi_buf.at[pl.ds(0, NLAST)],
                        vi_out.at[pl.ds(lo, NLAST)])

    def z_d(j, _):
        pltpu.make_async_copy(zbuf, msg_out.at[idst2.at[j]], sems).wait()
        return 0
    lax.fori_loop(0, nit, z_d, 0)


_k2 = pl.kernel(
    _body2,
    out_type=[
        jax.ShapeDtypeStruct((NNODES, ND), jnp.float32),
        jax.ShapeDtypeStruct((NNODES,), jnp.float32),
        jax.ShapeDtypeStruct((NNODES,), jnp.int32),
    ],
    mesh=plsc.VectorSubcoreMesh(core_axis_name="c", subcore_axis_name="s"),
    compiler_params=pltpu.CompilerParams(needs_layout_passes=False),
    scratch_types=[
        pltpu.VMEM((NT,), jnp.int32),
        pltpu.VMEM((NT,), jnp.int32),
        pltpu.VMEM((NT,), jnp.int32),
        pltpu.VMEM((NT,), jnp.int32),
        pltpu.VMEM((NT,), jnp.int32),
        pltpu.VMEM((NT,), jnp.int32),
        pltpu.VMEM((NT,), jnp.int32),
        pltpu.VMEM((NT,), jnp.int32),
        pltpu.VMEM((NT,), jnp.int32),
        pltpu.VMEM((16,), jnp.int32),
        pltpu.VMEM((TBUF,), jnp.int32),
        pltpu.VMEM((TBUF,), jnp.int32),
        pltpu.VMEM((TBUF,), jnp.int32),
        pltpu.VMEM((NTR, 128), jnp.int32),
        pltpu.VMEM((TBUF,), jnp.float32),
        pltpu.VMEM((128, ND), jnp.float32),
        pltpu.VMEM((128, ND), jnp.float32),
        pltpu.VMEM((128, ND), jnp.float32),
        pltpu.VMEM((128, ND), jnp.float32),
        pltpu.VMEM((128, ND), jnp.float32),
        pltpu.SemaphoreType.DMA,
        pltpu.SemaphoreType.DMA,
        pltpu.SemaphoreType.DMA,
        pltpu.SemaphoreType.DMA,
        pltpu.SemaphoreType.DMA,
        pltpu.SemaphoreType.DMA,
        pltpu.SemaphoreType.DMA,
        pltpu.SemaphoreType.DMA,
    ],
)


def kernel(node_ids, messages, timestamps, n_nodes):
    (partials,) = _k1(node_ids)
    zeros = jnp.zeros((128, ND), jnp.float32)
    nn = jnp.full((16,), n_nodes, jnp.int32)
    msg_o, ts_o, vi_o = _k2(partials, messages, timestamps, zeros, nn)
    return msg_o, ts_o, vi_o > 0
